# baseline (speedup 1.0000x reference)
import jax
import jax.numpy as jnp
from jax.experimental import pallas as pl


def _fused_kernel_t(xt_ref, w_ref, b_ref, o_ref):
    acc = jax.lax.dot_general(
        xt_ref[...], w_ref[...], (((0,), (0,)), ((), ())),
        preferred_element_type=jnp.float32,
    )
    o_ref[...] = jnp.maximum(acc + b_ref[...], 0.0)


@jax.jit
def _run(inputs, weights, bias2d):
    m, k = inputs.shape
    n = weights.shape[1]
    xt = inputs.T
    bn = 1024
    return pl.pallas_call(
        _fused_kernel_t,
        grid=(m // bn,),
        in_specs=[
            pl.BlockSpec((k, bn), lambda i: (0, i)),
            pl.BlockSpec((k, n), lambda i: (0, 0)),
            pl.BlockSpec((1, n), lambda i: (0, 0)),
        ],
        out_specs=pl.BlockSpec((bn, n), lambda i: (i, 0)),
        out_shape=jax.ShapeDtypeStruct((m, n), jnp.float32),
    )(xt, weights, bias2d)


def kernel(inputs, kernel, bias):
    return _run(inputs, kernel, bias.reshape(1, -1))


# speedup vs baseline: 1.1079x; 1.1079x over previous
import jax
import jax.numpy as jnp
from jax.experimental import pallas as pl


def _fused_kernel_t(xt_ref, w_ref, b_ref, o_ref):
    acc = jax.lax.dot_general(
        xt_ref[...], w_ref[...], (((0,), (0,)), ((), ())),
        preferred_element_type=jnp.float32,
    )
    o_ref[...] = jnp.maximum(acc + b_ref[...], 0.0)


@jax.jit
def _run(inputs, weights, bias2d):
    m, k = inputs.shape
    n = weights.shape[1]
    xt = inputs.T
    bn = 4096
    return pl.pallas_call(
        _fused_kernel_t,
        grid=(m // bn,),
        in_specs=[
            pl.BlockSpec((k, bn), lambda i: (0, i)),
            pl.BlockSpec((k, n), lambda i: (0, 0)),
            pl.BlockSpec((1, n), lambda i: (0, 0)),
        ],
        out_specs=pl.BlockSpec((bn, n), lambda i: (i, 0)),
        out_shape=jax.ShapeDtypeStruct((m, n), jnp.float32),
    )(xt, weights, bias2d)


def kernel(inputs, kernel, bias):
    return _run(inputs, kernel, bias.reshape(1, -1))
